# baseline (speedup 1.0000x reference)
import functools

import jax
import jax.numpy as jnp
from jax import lax
from jax.experimental import pallas as pl
from jax.experimental.pallas import tpu as pltpu
from jax.experimental.pallas import tpu_sc as plsc

NUM_CORES = 2
NUM_SUBCORES = 16
L = 16
NIDX = 2
NST = 4


@functools.lru_cache(maxsize=None)
def _make_gather(S, B, V, D):
    n_bhi = B // 128
    n_grp = B // L
    mesh = plsc.VectorSubcoreMesh(core_axis_name="c", subcore_axis_name="s")

    @functools.partial(
        pl.kernel,
        mesh=mesh,
        out_type=jax.ShapeDtypeStruct((S, D // 8, n_bhi, 8, 128), jnp.float32),
        scratch_types=[
            pltpu.VMEM((V,), jnp.float32),
            [pltpu.VMEM((B,), jnp.int32) for _ in range(NIDX)],
            [pltpu.VMEM((n_bhi, 128), jnp.float32) for _ in range(NST)],
            [pltpu.SemaphoreType.DMA for _ in range(NIDX)],
            [pltpu.SemaphoreType.DMA for _ in range(NST)],
        ],
        compiler_params=pltpu.CompilerParams(
            use_tc_tiling_on_sc=False, needs_layout_passes=False
        ),
    )
    def gather_kernel(xT_hbm, tabT_hbm, out_hbm, trow, idx4, stage2, isem, osem):
        wid = lax.axis_index("s") * NUM_CORES + lax.axis_index("c")
        c_hi = wid // 8
        c_lo = wid % 8
        for r in range(NIDX - 1):
            pltpu.async_copy(xT_hbm.at[r], idx4[r], isem[r])
        pltpu.sync_copy(tabT_hbm.at[wid], trow)

        def step(s, r, par):
            idx_v, stage_v = idx4[r], stage2[par]
            pltpu.make_async_copy(xT_hbm.at[s], idx_v, isem[r]).wait()

            @pl.when(s + NIDX - 1 < S)
            def _prefetch():
                pltpu.async_copy(
                    xT_hbm.at[s + NIDX - 1],
                    idx4[(r + NIDX - 1) % NIDX],
                    isem[(r + NIDX - 1) % NIDX],
                )

            @pl.when(s >= NST)
            def _drain():
                pltpu.make_async_copy(
                    stage_v, out_hbm.at[s - NST, c_hi, :, c_lo, :], osem[par]
                ).wait()

            @plsc.parallel_loop(0, n_grp, step=1, unroll=8)
            def _vg(g):
                idx16 = idx_v[pl.ds(g * L, L)]
                vals = plsc.load_gather(trow, [idx16])
                stage_v[g // 8, pl.ds((g % 8) * L, L)] = vals

            pltpu.async_copy(
                stage_v, out_hbm.at[s, c_hi, :, c_lo, :], osem[par]
            )

        PERIOD = 4
        def four(j, _):
            for q in range(PERIOD):
                step(PERIOD * j + q, q % NIDX, q % NST)
            return 0

        lax.fori_loop(0, S // PERIOD, four, 0)
        for s in range(S - NST, S):
            pltpu.make_async_copy(
                stage2[s % NST], out_hbm.at[s, c_hi, :, c_lo, :], osem[s % NST]
            ).wait()

    return gather_kernel


def kernel(x, table):
    B, S = x.shape
    V, D = table.shape
    xT = x.T.astype(jnp.int32)
    tabT = table.T
    out5 = _make_gather(S, B, V, D)(xT, tabT)
    return out5.transpose(2, 4, 0, 1, 3).reshape(B, S, D)


# speedup vs baseline: 1.3967x; 1.3967x over previous
import functools

import jax
import jax.numpy as jnp
from jax import lax
from jax.experimental import pallas as pl
from jax.experimental.pallas import tpu as pltpu
from jax.experimental.pallas import tpu_sc as plsc

NUM_CORES = 2
NUM_SUBCORES = 16
L = 16
NH = 6
NST = 4
PERIOD = 12


@functools.lru_cache(maxsize=None)
def _make_gather(S, B, V, D):
    n_bhi = B // 128
    n_grp = B // L
    H = B // 2
    mesh = plsc.VectorSubcoreMesh(core_axis_name="c", subcore_axis_name="s")

    @functools.partial(
        pl.kernel,
        mesh=mesh,
        out_type=jax.ShapeDtypeStruct((S, D // 8, n_bhi, 8, 128), jnp.float32),
        scratch_types=[
            pltpu.VMEM((V,), jnp.float32),
            [pltpu.VMEM((H,), jnp.int32) for _ in range(NH)],
            [pltpu.VMEM((n_bhi, 128), jnp.float32) for _ in range(NST)],
            [pltpu.SemaphoreType.DMA for _ in range(NH)],
            [pltpu.SemaphoreType.DMA for _ in range(NST)],
        ],
        compiler_params=pltpu.CompilerParams(
            use_tc_tiling_on_sc=False, needs_layout_passes=False
        ),
    )
    def gather_kernel(xT_hbm, tabT_hbm, out_hbm, trow, idxh, stage4, isem, osem):
        wid = lax.axis_index("s") * NUM_CORES + lax.axis_index("c")
        c_hi = wid // 8
        c_lo = wid % 8

        def fetch(slot_base, row_idx):
            row = xT_hbm.at[row_idx]
            for h in range(2):
                b = (2 * slot_base + h) % NH
                pltpu.async_copy(row.at[pl.ds(h * H, H)], idxh[b], isem[b])

        for r in range(2):
            fetch(r, r)
        pltpu.sync_copy(tabT_hbm.at[wid], trow)

        def step(s, q, do_prefetch, do_drain, s_ref=None):
            sv = s if s_ref is None else s_ref
            ha, hb = (2 * q) % NH, (2 * q + 1) % NH
            par = q % NST
            stage_v = stage4[par]
            pltpu.make_async_copy(
                xT_hbm.at[sv, pl.ds(0, H)], idxh[ha], isem[ha]
            ).wait()
            if do_prefetch:
                fetch(q + 2, sv + 2)
            if do_drain:
                pltpu.make_async_copy(
                    stage_v, out_hbm.at[sv - NST, c_hi, :, c_lo, :], osem[par]
                ).wait()

            @plsc.parallel_loop(0, n_grp // 2, step=1, unroll=8)
            def _vga(g):
                idx16 = idxh[ha][pl.ds(g * L, L)]
                vals = plsc.load_gather(trow, [idx16])
                stage_v[g // 8, pl.ds((g % 8) * L, L)] = vals

            pltpu.make_async_copy(
                xT_hbm.at[sv, pl.ds(H, H)], idxh[hb], isem[hb]
            ).wait()

            @plsc.parallel_loop(0, n_grp // 2, step=1, unroll=8)
            def _vgb(g):
                idx16 = idxh[hb][pl.ds(g * L, L)]
                vals = plsc.load_gather(trow, [idx16])
                stage_v[(n_bhi // 2) + g // 8, pl.ds((g % 8) * L, L)] = vals

            pltpu.async_copy(stage_v, out_hbm.at[sv, c_hi, :, c_lo, :], osem[par])

        for s in range(PERIOD):
            step(s, s, s + 2 < S, s >= NST)

        def body(j, _):
            s0 = PERIOD * j
            for q in range(PERIOD):
                step(q, q, True, True, s_ref=s0 + q)
            return 0

        lax.fori_loop(1, S // PERIOD, body, 0)

        for s in range(PERIOD * (S // PERIOD), S):
            step(s, s % PERIOD, s + 2 < S, True)

        for s in range(S - NST, S):
            par = (s % PERIOD) % NST
            pltpu.make_async_copy(
                stage4[par], out_hbm.at[s, c_hi, :, c_lo, :], osem[par]
            ).wait()

    return gather_kernel


def kernel(x, table):
    B, S = x.shape
    V, D = table.shape
    xT = x.T.astype(jnp.int32)
    tabT = table.T
    out5 = _make_gather(S, B, V, D)(xT, tabT)
    return out5.transpose(2, 4, 0, 1, 3).reshape(B, S, D)


# speedup vs baseline: 1.4765x; 1.0571x over previous
import functools

import jax
import jax.numpy as jnp
from jax import lax
from jax.experimental import pallas as pl
from jax.experimental.pallas import tpu as pltpu
from jax.experimental.pallas import tpu_sc as plsc

NUM_CORES = 2
NUM_SUBCORES = 16
L = 16
NIDX = 4


@functools.lru_cache(maxsize=None)
def _make_gather(S, B, V, D):
    n_bhi = B // 128
    n_grp = B // L
    mesh = plsc.VectorSubcoreMesh(core_axis_name="c", subcore_axis_name="s")

    @functools.partial(
        pl.kernel,
        mesh=mesh,
        out_type=jax.ShapeDtypeStruct((S, D // 8, n_bhi, 8, 128), jnp.float32),
        scratch_types=[
            pltpu.VMEM((V,), jnp.float32),
            [pltpu.VMEM((B,), jnp.int32) for _ in range(NIDX)],
            [pltpu.VMEM((n_bhi, 128), jnp.float32) for _ in range(2)],
            [pltpu.SemaphoreType.DMA for _ in range(NIDX)],
            [pltpu.SemaphoreType.DMA for _ in range(4)],
        ],
        compiler_params=pltpu.CompilerParams(
            use_tc_tiling_on_sc=False, needs_layout_passes=False
        ),
    )
    def gather_kernel(xT_hbm, tabT_hbm, out_hbm, trow, idx4, stage2, isem, osem):
        wid = lax.axis_index("s") * NUM_CORES + lax.axis_index("c")
        c_hi = wid // 8
        c_lo = wid % 8
        for r in range(NIDX - 1):
            pltpu.async_copy(xT_hbm.at[r], idx4[r], isem[r])
        pltpu.sync_copy(tabT_hbm.at[wid], trow)

        def step(s, r, par):
            idx_v, stage_v = idx4[r], stage2[par]
            pltpu.make_async_copy(xT_hbm.at[s], idx_v, isem[r]).wait()

            @pl.when(s + NIDX - 1 < S)
            def _prefetch():
                pltpu.async_copy(
                    xT_hbm.at[s + NIDX - 1],
                    idx4[(r + NIDX - 1) % NIDX],
                    isem[(r + NIDX - 1) % NIDX],
                )

            @pl.when(s >= 2)
            def _drain():
                for h in range(2):
                    pltpu.make_async_copy(
                        stage_v.at[pl.ds(h * 16, 16)],
                        out_hbm.at[s - 2, c_hi, pl.ds(h * 16, 16), c_lo, :],
                        osem[2 * par + h],
                    ).wait()

            @plsc.parallel_loop(0, n_grp, step=1, unroll=8)
            def _vg(g):
                idx16 = idx_v[pl.ds(g * L, L)]
                vals = plsc.load_gather(trow, [idx16])
                stage_v[g // 8, pl.ds((g % 8) * L, L)] = vals

            for h in range(2):
                pltpu.async_copy(
                    stage_v.at[pl.ds(h * 16, 16)],
                    out_hbm.at[s, c_hi, pl.ds(h * 16, 16), c_lo, :],
                    osem[2 * par + h],
                )

        def four(j, _):
            for q in range(NIDX):
                step(NIDX * j + q, q, q % 2)
            return 0

        lax.fori_loop(0, S // NIDX, four, 0)
        for par, s in ((0, S - 2), (1, S - 1)):
            for h in range(2):
                pltpu.make_async_copy(
                    stage2[par].at[pl.ds(h * 16, 16)],
                    out_hbm.at[s, c_hi, pl.ds(h * 16, 16), c_lo, :],
                    osem[2 * par + h],
                ).wait()

    return gather_kernel


def kernel(x, table):
    B, S = x.shape
    V, D = table.shape
    xT = x.T.astype(jnp.int32)
    tabT = table.T
    out5 = _make_gather(S, B, V, D)(xT, tabT)
    return out5.transpose(2, 4, 0, 1, 3).reshape(B, S, D)


# speedup vs baseline: 1.4872x; 1.0073x over previous
import functools

import jax
import jax.numpy as jnp
from jax import lax
from jax.experimental import pallas as pl
from jax.experimental.pallas import tpu as pltpu
from jax.experimental.pallas import tpu_sc as plsc

NUM_CORES = 2
NUM_SUBCORES = 16
L = 16
NIDX = 4


@functools.lru_cache(maxsize=None)
def _make_gather(S, B, V, D):
    n_bhi = B // 128
    n_grp = B // L
    mesh = plsc.VectorSubcoreMesh(core_axis_name="c", subcore_axis_name="s")

    @functools.partial(
        pl.kernel,
        mesh=mesh,
        out_type=jax.ShapeDtypeStruct((S, D // 8, n_bhi, 8, 128), jnp.float32),
        scratch_types=[
            pltpu.VMEM((V,), jnp.float32),
            [pltpu.VMEM((B,), jnp.int32) for _ in range(NIDX)],
            [pltpu.VMEM((n_bhi, 128), jnp.float32) for _ in range(2)],
            [pltpu.SemaphoreType.DMA for _ in range(NIDX)],
            [pltpu.SemaphoreType.DMA for _ in range(2)],
        ],
        compiler_params=pltpu.CompilerParams(
            use_tc_tiling_on_sc=False, needs_layout_passes=False
        ),
    )
    def gather_kernel(xT_hbm, tabT_hbm, out_hbm, trow, idx4, stage2, isem, osem):
        wid = lax.axis_index("s") * NUM_CORES + lax.axis_index("c")
        c_hi = wid // 8
        c_lo = wid % 8
        for r in range(NIDX - 1):
            pltpu.async_copy(xT_hbm.at[r], idx4[r], isem[r])
        pltpu.sync_copy(tabT_hbm.at[wid], trow)

        def step(s, r, par):
            idx_v, stage_v = idx4[r], stage2[par]
            pltpu.make_async_copy(xT_hbm.at[s], idx_v, isem[r]).wait()

            @pl.when(s + NIDX - 1 < S)
            def _prefetch():
                pltpu.async_copy(
                    xT_hbm.at[s + NIDX - 1],
                    idx4[(r + NIDX - 1) % NIDX],
                    isem[(r + NIDX - 1) % NIDX],
                )

            @pl.when(s >= 2)
            def _drain():
                pltpu.make_async_copy(
                    stage_v, out_hbm.at[s - 2, c_hi, :, c_lo, :], osem[par]
                ).wait()

            @plsc.parallel_loop(0, n_grp, step=1, unroll=16)
            def _vg(g):
                idx16 = idx_v[pl.ds(g * L, L)]
                vals = plsc.load_gather(trow, [idx16])
                stage_v[g // 8, pl.ds((g % 8) * L, L)] = vals

            pltpu.async_copy(
                stage_v, out_hbm.at[s, c_hi, :, c_lo, :], osem[par]
            )

        def four(j, _):
            for q in range(NIDX):
                step(NIDX * j + q, q, q % 2)
            return 0

        lax.fori_loop(0, S // NIDX, four, 0)
        for par, s in ((0, S - 2), (1, S - 1)):
            pltpu.make_async_copy(
                stage2[par], out_hbm.at[s, c_hi, :, c_lo, :], osem[par]
            ).wait()

    return gather_kernel


def kernel(x, table):
    B, S = x.shape
    V, D = table.shape
    xT = x.T.astype(jnp.int32)
    tabT = table.T
    out5 = _make_gather(S, B, V, D)(xT, tabT)
    return out5.transpose(2, 4, 0, 1, 3).reshape(B, S, D)


# speedup vs baseline: 1.4889x; 1.0012x over previous
import functools

import jax
import jax.numpy as jnp
from jax import lax
from jax.experimental import pallas as pl
from jax.experimental.pallas import tpu as pltpu
from jax.experimental.pallas import tpu_sc as plsc

NUM_CORES = 2
NUM_SUBCORES = 16
L = 16
NIDX = 4


@functools.lru_cache(maxsize=None)
def _make_gather(S, B, V, D):
    n_bhi = B // 128
    n_grp = B // L
    mesh = plsc.VectorSubcoreMesh(core_axis_name="c", subcore_axis_name="s")

    @functools.partial(
        pl.kernel,
        mesh=mesh,
        out_type=jax.ShapeDtypeStruct((S, D // 8, n_bhi, 8, 128), jnp.float32),
        scratch_types=[
            pltpu.VMEM((V,), jnp.float32),
            [pltpu.VMEM((B,), jnp.int32) for _ in range(NIDX)],
            [pltpu.VMEM((n_bhi, 128), jnp.float32) for _ in range(2)],
            [pltpu.SemaphoreType.DMA for _ in range(NIDX)],
            [pltpu.SemaphoreType.DMA for _ in range(2)],
        ],
        compiler_params=pltpu.CompilerParams(
            use_tc_tiling_on_sc=False, needs_layout_passes=False
        ),
    )
    def gather_kernel(xT_hbm, tabT_hbm, out_hbm, trow, idx4, stage2, isem, osem):
        wid = lax.axis_index("s") * NUM_CORES + lax.axis_index("c")
        c_hi = wid // 8
        c_lo = wid % 8
        for r in range(NIDX - 1):
            pltpu.async_copy(xT_hbm.at[r], idx4[r], isem[r])
        pltpu.sync_copy(tabT_hbm.at[wid], trow)

        def step(s, r, par):
            idx_v, stage_v = idx4[r], stage2[par]
            pltpu.make_async_copy(xT_hbm.at[s], idx_v, isem[r]).wait()

            @pl.when(s + NIDX - 1 < S)
            def _prefetch():
                pltpu.async_copy(
                    xT_hbm.at[s + NIDX - 1],
                    idx4[(r + NIDX - 1) % NIDX],
                    isem[(r + NIDX - 1) % NIDX],
                )

            @pl.when(s >= 2)
            def _drain():
                pltpu.make_async_copy(
                    stage_v, out_hbm.at[s - 2, c_hi, :, c_lo, :], osem[par]
                ).wait()

            @plsc.parallel_loop(0, n_grp, step=1, unroll=8)
            def _vg(g):
                idx16 = idx_v[pl.ds(g * L, L)]
                vals = plsc.load_gather(trow, [idx16])
                stage_v[g // 8, pl.ds((g % 8) * L, L)] = vals

            pltpu.async_copy(
                stage_v, out_hbm.at[s, c_hi, :, c_lo, :], osem[par]
            )

        def four(j, _):
            for q in range(NIDX):
                step(NIDX * j + q, q, q % 2)
            return 0

        lax.fori_loop(0, S // NIDX, four, 0)
        for par, s in ((0, S - 2), (1, S - 1)):
            pltpu.make_async_copy(
                stage2[par], out_hbm.at[s, c_hi, :, c_lo, :], osem[par]
            ).wait()

    return gather_kernel


def kernel(x, table):
    B, S = x.shape
    V, D = table.shape
    xT = x.T.astype(jnp.int32)
    tabT = table.T
    out5 = _make_gather(S, B, V, D)(xT, tabT)
    return out5.transpose(2, 4, 0, 1, 3).reshape(B, S, D)
